# baseline (speedup 1.0000x reference)
import jax
import jax.numpy as jnp
from jax.experimental import pallas as pl
from jax.experimental.pallas import tpu as pltpu

SHIFT = 8192


def _dma_body(length, onehot_ref, stem_ref, sel_ref, wav_ref, sem_wav, sem_sel):
    B, S = onehot_ref.shape

    idxs = []
    for b in range(B):
        idx = jnp.int32(0)
        best = onehot_ref[b, 0]
        for j in range(1, S):
            v = onehot_ref[b, j]
            gt = v > best
            idx = jnp.where(gt, jnp.int32(j), idx)
            best = jnp.where(gt, v, best)
        idxs.append(idx)

    for b in range(B):
        pltpu.make_async_copy(
            stem_ref.at[b, :, :, :length], wav_ref.at[b], sem_wav
        ).start()
        pltpu.make_async_copy(
            stem_ref.at[b, idxs[b], :, :length], sel_ref.at[b], sem_sel
        ).start()
    for b in range(B):
        pltpu.make_async_copy(
            stem_ref.at[b, :, :, :length], wav_ref.at[b], sem_wav
        ).wait()
        pltpu.make_async_copy(
            stem_ref.at[b, 0, :, :length], sel_ref.at[b], sem_sel
        ).wait()


def kernel(selected_stem, one_hot_vector, stem_data):
    B, S, C, T = stem_data.shape
    length = T - SHIFT

    sel, wav = pl.pallas_call(
        lambda *refs: _dma_body(length, *refs),
        in_specs=[
            pl.BlockSpec(memory_space=pltpu.MemorySpace.SMEM),
            pl.BlockSpec(memory_space=pltpu.MemorySpace.HBM),
        ],
        out_specs=[
            pl.BlockSpec(memory_space=pltpu.MemorySpace.HBM),
            pl.BlockSpec(memory_space=pltpu.MemorySpace.HBM),
        ],
        out_shape=[
            jax.ShapeDtypeStruct((B, C, length), stem_data.dtype),
            jax.ShapeDtypeStruct((B, S, C, length), stem_data.dtype),
        ],
        scratch_shapes=[pltpu.SemaphoreType.DMA, pltpu.SemaphoreType.DMA],
    )(one_hot_vector, stem_data)

    return (sel, one_hot_vector, wav)


# speedup vs baseline: 10.0603x; 10.0603x over previous
import functools
import jax
import jax.numpy as jnp
from jax import lax
from jax.experimental import pallas as pl
from jax.experimental.pallas import tpu as pltpu
from jax.experimental.pallas import tpu_sc as plsc

SHIFT = 8192
VROW = 2048
NW = 32


def _wav_body(stem_ref, wav_ref):
    wav_ref[...] = stem_ref[...]


def _make_sel_kernel(B, S, C, T, length):
    kt = T // VROW
    kl = length // VROW
    n_pairs = C * kl
    per_w = -(-n_pairs // NW)
    mesh = plsc.VectorSubcoreMesh(core_axis_name="c", subcore_axis_name="s")

    @functools.partial(
        pl.kernel,
        mesh=mesh,
        out_type=jax.ShapeDtypeStruct((B * C * kl, VROW), jnp.float32),
        scratch_types=[
            pltpu.VMEM((S * 16,), jnp.float32),
            pltpu.VMEM((16, VROW), jnp.float32),
            pltpu.VMEM((16, VROW), jnp.float32),
            pltpu.SemaphoreType.DMA,
            pltpu.SemaphoreType.DMA,
        ],
    )
    def sel_kernel(oh_hbm, stem_hbm, sel_hbm, oh_v, buf0, buf1, sem_in, sem_out):
        wid = lax.axis_index("s") * 2 + lax.axis_index("c")

        pltpu.sync_copy(oh_hbm, oh_v)
        lanes = lax.iota(jnp.int32, 16)
        best = oh_v[pl.ds(0, 16)]
        idx_vec = jnp.zeros((16,), jnp.int32)
        for s in range(1, S):
            v = oh_v[pl.ds(s * 16, 16)]
            gt = v > best
            idx_vec = jnp.where(gt, jnp.int32(s), idx_vec)
            best = jnp.where(gt, v, best)

        gbase = lanes * (S * C * kt) + idx_vec * (C * kt)
        sbase = lanes * (C * kl)

        bufs = (buf0, buf1)
        pairs = []
        for j in range(per_w):
            p = wid + j * NW
            p = jnp.minimum(p, n_pairs - 1)
            c = p % C
            k = p // C
            pairs.append((c, k))

        for j, (c, k) in enumerate(pairs):
            buf = bufs[j % 2]
            if j >= 2:
                cp, kp = pairs[j - 2]
                pltpu.make_async_copy(
                    buf, sel_hbm.at[sbase + (cp * kl + kp)], sem_out
                ).wait()
            pltpu.async_copy(
                stem_hbm.at[gbase + (c * kt + k)], buf, sem_in
            ).wait()
            pltpu.make_async_copy(
                buf, sel_hbm.at[sbase + (c * kl + k)], sem_out
            ).start()
        for j in (per_w - 2, per_w - 1):
            c, k = pairs[j]
            pltpu.make_async_copy(
                bufs[j % 2], sel_hbm.at[sbase + (c * kl + k)], sem_out
            ).wait()

    return sel_kernel


def kernel(selected_stem, one_hot_vector, stem_data):
    B, S, C, T = stem_data.shape
    length = T - SHIFT

    sel = _make_sel_kernel(B, S, C, T, length)(
        one_hot_vector.T.reshape(S * B),
        stem_data.reshape(B * S * C * (T // VROW), VROW),
    ).reshape(B, C, length)

    wav = pl.pallas_call(
        _wav_body,
        grid=(B,),
        in_specs=[pl.BlockSpec((1, S, C, length), lambda b: (b, 0, 0, 0))],
        out_specs=pl.BlockSpec((1, S, C, length), lambda b: (b, 0, 0, 0)),
        out_shape=jax.ShapeDtypeStruct((B, S, C, length), stem_data.dtype),
        compiler_params=pltpu.CompilerParams(
            dimension_semantics=("parallel",),
        ),
    )(stem_data)

    return (sel, one_hot_vector, wav)


# speedup vs baseline: 40.9398x; 4.0694x over previous
import functools
import jax
import jax.numpy as jnp
from jax import lax
from jax.experimental import pallas as pl
from jax.experimental.pallas import tpu as pltpu
from jax.experimental.pallas import tpu_sc as plsc

SHIFT = 8192
CHUNK = 2048
NW = 32


def _wav_body(stem_ref, wav_ref):
    wav_ref[...] = stem_ref[...]


def _make_sel_kernel(B, S, C, T, length):
    kl = length // CHUNK
    n_pairs = C * kl
    per_w = -(-n_pairs // NW)
    mesh = plsc.VectorSubcoreMesh(core_axis_name="c", subcore_axis_name="s")

    @functools.partial(
        pl.kernel,
        mesh=mesh,
        out_type=jax.ShapeDtypeStruct((B, C, length), jnp.float32),
        scratch_types=[
            pltpu.VMEM((S * 16,), jnp.float32),
            pltpu.VMEM((16, CHUNK), jnp.float32),
            pltpu.VMEM((16, CHUNK), jnp.float32),
            pltpu.SemaphoreType.DMA,
            pltpu.SemaphoreType.DMA,
        ],
    )
    def sel_kernel(oh_hbm, stem_hbm, sel_hbm, oh_v, buf0, buf1, sem_in, sem_out):
        wid = lax.axis_index("s") * 2 + lax.axis_index("c")
        stem2 = stem_hbm.reshape(B * S * C, T)
        sel2 = sel_hbm.reshape(B * C, length)

        pltpu.sync_copy(oh_hbm, oh_v)
        lanes = lax.iota(jnp.int32, 16)
        best = oh_v[pl.ds(0, 16)]
        idx_vec = jnp.zeros((16,), jnp.int32)
        for s in range(1, S):
            v = oh_v[pl.ds(s * 16, 16)]
            gt = v > best
            idx_vec = jnp.where(gt, jnp.int32(s), idx_vec)
            best = jnp.where(gt, v, best)

        grow = lanes * (S * C) + idx_vec * C
        srow = lanes * C

        bufs = (buf0, buf1)
        pairs = []
        for j in range(per_w):
            p = wid + j * NW
            p = jnp.minimum(p, n_pairs - 1)
            pairs.append((p % C, p // C))

        for j, (c, k) in enumerate(pairs):
            buf = bufs[j % 2]
            if j >= 2:
                cp, kp = pairs[j - 2]
                pltpu.make_async_copy(
                    buf, sel2.at[srow + cp, pl.ds(kp * CHUNK, CHUNK)], sem_out
                ).wait()
            pltpu.async_copy(
                stem2.at[grow + c, pl.ds(k * CHUNK, CHUNK)], buf, sem_in
            ).wait()
            pltpu.make_async_copy(
                buf, sel2.at[srow + c, pl.ds(k * CHUNK, CHUNK)], sem_out
            ).start()
        for j in (per_w - 2, per_w - 1):
            c, k = pairs[j]
            pltpu.make_async_copy(
                bufs[j % 2], sel2.at[srow + c, pl.ds(k * CHUNK, CHUNK)], sem_out
            ).wait()

    return sel_kernel


def kernel(selected_stem, one_hot_vector, stem_data):
    B, S, C, T = stem_data.shape
    length = T - SHIFT

    sel = _make_sel_kernel(B, S, C, T, length)(
        one_hot_vector.T.reshape(S * B), stem_data
    )

    wav = pl.pallas_call(
        _wav_body,
        grid=(B,),
        in_specs=[pl.BlockSpec((1, S, C, length), lambda b: (b, 0, 0, 0))],
        out_specs=pl.BlockSpec((1, S, C, length), lambda b: (b, 0, 0, 0)),
        out_shape=jax.ShapeDtypeStruct((B, S, C, length), stem_data.dtype),
        compiler_params=pltpu.CompilerParams(
            dimension_semantics=("parallel",),
        ),
    )(stem_data)

    return (sel, one_hot_vector, wav)


# speedup vs baseline: 53.0531x; 1.2959x over previous
import jax
import jax.numpy as jnp
from jax.experimental import pallas as pl
from jax.experimental.pallas import tpu as pltpu

SHIFT = 8192


def _shift_body(onehot_ref, stem_ref, sel_ref, wav_ref):
    b = pl.program_id(0)
    row = onehot_ref[pl.ds(b, 1), :]
    idx = jnp.argmax(row)
    blk = stem_ref[...]
    wav_ref[...] = blk
    sel_ref[...] = stem_ref[0, pl.ds(idx, 1), :, :]


def kernel(selected_stem, one_hot_vector, stem_data):
    B, S, C, T = stem_data.shape
    length = T - SHIFT
    BT = length
    grid = (B, length // BT)

    sel, wav = pl.pallas_call(
        _shift_body,
        grid=grid,
        in_specs=[
            pl.BlockSpec((B, S), lambda b, t: (0, 0)),
            pl.BlockSpec((1, S, C, BT), lambda b, t: (b, 0, 0, t)),
        ],
        out_specs=[
            pl.BlockSpec((1, C, BT), lambda b, t: (b, 0, t)),
            pl.BlockSpec((1, S, C, BT), lambda b, t: (b, 0, 0, t)),
        ],
        out_shape=[
            jax.ShapeDtypeStruct((B, C, length), stem_data.dtype),
            jax.ShapeDtypeStruct((B, S, C, length), stem_data.dtype),
        ],
        compiler_params=pltpu.CompilerParams(
            dimension_semantics=("parallel", "parallel"),
        ),
    )(one_hot_vector, stem_data)

    return (sel, one_hot_vector, wav)
